# baseline (speedup 1.0000x reference)
import functools

import jax
import jax.numpy as jnp
from jax import lax
from jax.experimental import pallas as pl
from jax.experimental.pallas import tpu as pltpu
from jax.experimental.pallas import tpu_sc as plsc

L = 16
D = 32
B = 16384
NC = 2
NS = 16
NW = NC * NS
BPW = B // NW
CHUNK = 128
NCHUNK = BPW // CHUNK
GROUPS = BPW // L


def _sc_body(uid_hbm, iid_hbm, ut_hbm, it_hbm, out_hbm,
             uidx_v, iidx_v, urows_v, irows_v, out_v, sem):
    wid = lax.axis_index("s") * NC + lax.axis_index("c")
    base = wid * BPW

    row0 = wid * NCHUNK
    pltpu.sync_copy(uid_hbm.at[pl.ds(row0, NCHUNK)], uidx_v)
    pltpu.sync_copy(iid_hbm.at[pl.ds(row0, NCHUNK)], iidx_v)

    copies = []
    for j in range(NCHUNK):
        copies.append(pltpu.async_copy(
            ut_hbm.at[uidx_v.at[j]],
            urows_v.at[pl.ds(j * CHUNK, CHUNK)], sem))
        copies.append(pltpu.async_copy(
            it_hbm.at[iidx_v.at[j]],
            irows_v.at[pl.ds(j * CHUNK, CHUNK)], sem))
    for c in copies:
        c.wait()

    iota = lax.iota(jnp.int32, L)

    def group(g, carry):
        row = g * L + iota
        acc = jnp.zeros((L,), jnp.float32)
        for d in range(D):
            col = lax.rem(iota + d, D)
            u = plsc.load_gather(urows_v, [row, col])
            v = plsc.load_gather(irows_v, [row, col])
            acc = acc + u * v
        out_v[pl.ds(g * L, L)] = acc
        return carry

    lax.fori_loop(0, GROUPS, group, 0)

    pltpu.sync_copy(out_v, out_hbm.at[pl.ds(base, BPW)])


@jax.jit
def _run(user_ids2d, item_ids2d, user_table, item_table):
    k = pl.kernel(
        _sc_body,
        out_type=jax.ShapeDtypeStruct((B,), jnp.float32),
        mesh=plsc.VectorSubcoreMesh(core_axis_name="c", subcore_axis_name="s"),
        compiler_params=pltpu.CompilerParams(
            needs_layout_passes=False, use_tc_tiling_on_sc=False),
        scratch_types=[
            pltpu.VMEM((NCHUNK, CHUNK), jnp.int32),
            pltpu.VMEM((NCHUNK, CHUNK), jnp.int32),
            pltpu.VMEM((BPW, D), jnp.float32),
            pltpu.VMEM((BPW, D), jnp.float32),
            pltpu.VMEM((BPW,), jnp.float32),
            pltpu.SemaphoreType.DMA,
        ],
    )
    return k(user_ids2d, item_ids2d, user_table, item_table)


def kernel(user_ids, item_ids, user_table, item_table):
    uid2d = user_ids.reshape(B // CHUNK, CHUNK)
    iid2d = item_ids.reshape(B // CHUNK, CHUNK)
    return _run(uid2d, iid2d, user_table, item_table)


# speedup vs baseline: 3.9292x; 3.9292x over previous
import functools

import jax
import jax.numpy as jnp
from jax import lax
from jax.experimental import pallas as pl
from jax.experimental.pallas import tpu as pltpu
from jax.experimental.pallas import tpu_sc as plsc

L = 16
D = 32
B = 16384
NC = 2
NS = 16
NW = NC * NS
BPW = B // NW
WAVE = 4
WPT = BPW // WAVE


def _sc_body(uid_hbm, iid_hbm, ut_hbm, it_hbm, out_hbm,
             uidx_v, iidx_v, ubuf_v, ibuf_v, out_v, usem, isem):
    wid = lax.axis_index("s") * NC + lax.axis_index("c")
    base = wid * BPW

    pltpu.sync_copy(uid_hbm.at[pl.ds(base, BPW)], uidx_v)
    pltpu.sync_copy(iid_hbm.at[pl.ds(base, BPW)], iidx_v)

    iota = lax.iota(jnp.int32, L)

    def vecs(w):
        g = (w // 4) * L
        return uidx_v[pl.ds(g, L)], iidx_v[pl.ds(g, L)]

    def extract(vec, l):
        return lax.reduce_max(jnp.where(iota == l, vec, 0), (0,))

    def fire(w):
        uvec, ivec = vecs(w)
        p = w % 2
        for s in range(WAVE):
            l = (w % 4) * WAVE + s
            for vec, tab, buf, sem in ((uvec, ut_hbm, ubuf_v, usem),
                                       (ivec, it_hbm, ibuf_v, isem)):
                sid = extract(vec, l)
                jb = pl.multiple_of((sid >> 7) << 7, 128)
                pltpu.async_copy(tab.at[:, pl.ds(jb, 128)], buf.at[p, s], sem)

    def wait_wave():
        for s in range(WAVE):
            pltpu.make_async_copy(ut_hbm.at[:, pl.ds(0, 128)],
                                  ubuf_v.at[0, s], usem).wait()
            pltpu.make_async_copy(it_hbm.at[:, pl.ds(0, 128)],
                                  ibuf_v.at[0, s], isem).wait()

    d_lo = iota
    d_hi = iota + L

    def compute(w):
        uvec, ivec = vecs(w)
        p = w % 2
        pb = jnp.full((L,), 0, jnp.int32) + p
        for s in range(WAVE):
            l = (w % 4) * WAVE + s
            sb = jnp.full((L,), s, jnp.int32)
            usid = extract(uvec, l)
            isid = extract(ivec, l)
            ulane = jnp.full((L,), 0, jnp.int32) + (usid & 127)
            ilane = jnp.full((L,), 0, jnp.int32) + (isid & 127)
            u_lo = plsc.load_gather(ubuf_v, [pb, sb, d_lo, ulane])
            u_hi = plsc.load_gather(ubuf_v, [pb, sb, d_hi, ulane])
            i_lo = plsc.load_gather(ibuf_v, [pb, sb, d_lo, ilane])
            i_hi = plsc.load_gather(ibuf_v, [pb, sb, d_hi, ilane])
            prod = u_lo * i_lo + u_hi * i_hi
            score = lax.reduce_sum(prod, (0,))
            k = jnp.full((L,), 0, jnp.int32) + (w * WAVE + s)
            plsc.store_scatter(out_v, [k],
                               jnp.full((L,), 0.0, jnp.float32) + score,
                               mask=iota == 0)

    fire(0)

    def body(w, carry):
        @pl.when(w + 1 < WPT)
        def _():
            fire(w + 1)
        wait_wave()
        compute(w)
        return carry

    lax.fori_loop(0, WPT, body, 0)

    pltpu.sync_copy(out_v, out_hbm.at[pl.ds(base, BPW)])


@jax.jit
def _run(user_ids, item_ids, user_table_t, item_table_t):
    k = pl.kernel(
        _sc_body,
        out_type=jax.ShapeDtypeStruct((B,), jnp.float32),
        mesh=plsc.VectorSubcoreMesh(core_axis_name="c", subcore_axis_name="s"),
        compiler_params=pltpu.CompilerParams(needs_layout_passes=False),
        scratch_types=[
            pltpu.VMEM((BPW,), jnp.int32),
            pltpu.VMEM((BPW,), jnp.int32),
            pltpu.VMEM((2, WAVE, D, 128), jnp.float32),
            pltpu.VMEM((2, WAVE, D, 128), jnp.float32),
            pltpu.VMEM((BPW,), jnp.float32),
            pltpu.SemaphoreType.DMA,
            pltpu.SemaphoreType.DMA,
        ],
    )
    return k(user_ids, item_ids, user_table_t, item_table_t)


def kernel(user_ids, item_ids, user_table, item_table):
    return _run(user_ids, item_ids, user_table.T, item_table.T)


# speedup vs baseline: 4.3263x; 1.1011x over previous
import functools

import jax
import jax.numpy as jnp
from jax import lax
from jax.experimental import pallas as pl
from jax.experimental.pallas import tpu as pltpu
from jax.experimental.pallas import tpu_sc as plsc

L = 16
D = 32
B = 16384
NC = 2
NS = 16
NW = NC * NS
BPW = B // NW
WAVE = 4
WPT = BPW // WAVE
DEPTH = 3


def _sc_body(uid_hbm, iid_hbm, ut_hbm, it_hbm, out_hbm,
             uidx_v, iidx_v, ubuf_v, ibuf_v, out_v, usem, isem):
    wid = lax.axis_index("s") * NC + lax.axis_index("c")
    base = wid * BPW

    pltpu.sync_copy(uid_hbm.at[pl.ds(base, BPW)], uidx_v)
    pltpu.sync_copy(iid_hbm.at[pl.ds(base, BPW)], iidx_v)

    iota = lax.iota(jnp.int32, L)

    def vecs(w):
        g = (w // 4) * L
        return uidx_v[pl.ds(g, L)], iidx_v[pl.ds(g, L)]

    def extract(vec, l):
        return lax.reduce_max(jnp.where(iota == l, vec, 0), (0,))

    def fire(w):
        uvec, ivec = vecs(w)
        p = w % DEPTH
        for s in range(WAVE):
            l = (w % 4) * WAVE + s
            for vec, tab, buf, sem in ((uvec, ut_hbm, ubuf_v, usem),
                                       (ivec, it_hbm, ibuf_v, isem)):
                sid = extract(vec, l)
                jb = pl.multiple_of((sid >> 7) << 7, 128)
                pltpu.async_copy(tab.at[:, pl.ds(jb, 128)], buf.at[p, s], sem)

    def wait_wave():
        for s in range(WAVE):
            pltpu.make_async_copy(ut_hbm.at[:, pl.ds(0, 128)],
                                  ubuf_v.at[0, s], usem).wait()
            pltpu.make_async_copy(it_hbm.at[:, pl.ds(0, 128)],
                                  ibuf_v.at[0, s], isem).wait()

    d_lo = iota
    d_hi = iota + L

    def compute(w):
        uvec, ivec = vecs(w)
        p = w % DEPTH
        pb = jnp.full((L,), 0, jnp.int32) + p
        for s in range(WAVE):
            l = (w % 4) * WAVE + s
            sb = jnp.full((L,), s, jnp.int32)
            usid = extract(uvec, l)
            isid = extract(ivec, l)
            ulane = jnp.full((L,), 0, jnp.int32) + (usid & 127)
            ilane = jnp.full((L,), 0, jnp.int32) + (isid & 127)
            u_lo = plsc.load_gather(ubuf_v, [pb, sb, d_lo, ulane])
            u_hi = plsc.load_gather(ubuf_v, [pb, sb, d_hi, ulane])
            i_lo = plsc.load_gather(ibuf_v, [pb, sb, d_lo, ilane])
            i_hi = plsc.load_gather(ibuf_v, [pb, sb, d_hi, ilane])
            prod = u_lo * i_lo + u_hi * i_hi
            score = lax.reduce_sum(prod, (0,))
            k = jnp.full((L,), 0, jnp.int32) + (w * WAVE + s)
            plsc.store_scatter(out_v, [k],
                               jnp.full((L,), 0.0, jnp.float32) + score,
                               mask=iota == 0)

    for w0 in range(DEPTH - 1):
        fire(w0)

    def body(w, carry):
        @pl.when(w + DEPTH - 1 < WPT)
        def _():
            fire(w + DEPTH - 1)
        wait_wave()
        compute(w)
        return carry

    lax.fori_loop(0, WPT, body, 0)

    pltpu.sync_copy(out_v, out_hbm.at[pl.ds(base, BPW)])


@jax.jit
def _run(user_ids, item_ids, user_table_t, item_table_t):
    k = pl.kernel(
        _sc_body,
        out_type=jax.ShapeDtypeStruct((B,), jnp.float32),
        mesh=plsc.VectorSubcoreMesh(core_axis_name="c", subcore_axis_name="s"),
        compiler_params=pltpu.CompilerParams(needs_layout_passes=False),
        scratch_types=[
            pltpu.VMEM((BPW,), jnp.int32),
            pltpu.VMEM((BPW,), jnp.int32),
            pltpu.VMEM((DEPTH, WAVE, D, 128), jnp.float32),
            pltpu.VMEM((DEPTH, WAVE, D, 128), jnp.float32),
            pltpu.VMEM((BPW,), jnp.float32),
            pltpu.SemaphoreType.DMA,
            pltpu.SemaphoreType.DMA,
        ],
    )
    return k(user_ids, item_ids, user_table_t, item_table_t)


def kernel(user_ids, item_ids, user_table, item_table):
    return _run(user_ids, item_ids, user_table.T, item_table.T)


# speedup vs baseline: 4.4039x; 1.0179x over previous
import functools

import jax
import jax.numpy as jnp
from jax import lax
from jax.experimental import pallas as pl
from jax.experimental.pallas import tpu as pltpu
from jax.experimental.pallas import tpu_sc as plsc

L = 16
D = 32
B = 16384
NC = 2
NS = 16
NW = NC * NS
BPW = B // NW
WAVE = 2
WPT = BPW // WAVE
DEPTH = 7
WPG = L // WAVE


def _sc_body(uid_hbm, iid_hbm, ut_hbm, it_hbm, out_hbm,
             uidx_v, iidx_v, ubuf_v, ibuf_v, out_v, usem, isem):
    wid = lax.axis_index("s") * NC + lax.axis_index("c")
    base = wid * BPW

    pltpu.sync_copy(uid_hbm.at[pl.ds(base, BPW)], uidx_v)
    pltpu.sync_copy(iid_hbm.at[pl.ds(base, BPW)], iidx_v)

    iota = lax.iota(jnp.int32, L)

    def vecs(w):
        g = (w // WPG) * L
        return uidx_v[pl.ds(g, L)], iidx_v[pl.ds(g, L)]

    def extract(vec, l):
        return lax.reduce_max(jnp.where(iota == l, vec, 0), (0,))

    def fire(w):
        uvec, ivec = vecs(w)
        p = w % DEPTH
        for s in range(WAVE):
            l = (w % WPG) * WAVE + s
            for vec, tab, buf, sem in ((uvec, ut_hbm, ubuf_v, usem),
                                       (ivec, it_hbm, ibuf_v, isem)):
                sid = extract(vec, l)
                jb = pl.multiple_of((sid >> 7) << 7, 128)
                pltpu.async_copy(tab.at[:, pl.ds(jb, 128)], buf.at[p, s], sem)

    def wait_wave():
        for s in range(WAVE):
            pltpu.make_async_copy(ut_hbm.at[:, pl.ds(0, 128)],
                                  ubuf_v.at[0, s], usem).wait()
            pltpu.make_async_copy(it_hbm.at[:, pl.ds(0, 128)],
                                  ibuf_v.at[0, s], isem).wait()

    d_lo = iota
    d_hi = iota + L

    def compute(w):
        uvec, ivec = vecs(w)
        p = w % DEPTH
        pb = jnp.full((L,), 0, jnp.int32) + p
        for s in range(WAVE):
            l = (w % WPG) * WAVE + s
            sb = jnp.full((L,), s, jnp.int32)
            usid = extract(uvec, l)
            isid = extract(ivec, l)
            ulane = jnp.full((L,), 0, jnp.int32) + (usid & 127)
            ilane = jnp.full((L,), 0, jnp.int32) + (isid & 127)
            u_lo = plsc.load_gather(ubuf_v, [pb, sb, d_lo, ulane])
            u_hi = plsc.load_gather(ubuf_v, [pb, sb, d_hi, ulane])
            i_lo = plsc.load_gather(ibuf_v, [pb, sb, d_lo, ilane])
            i_hi = plsc.load_gather(ibuf_v, [pb, sb, d_hi, ilane])
            prod = u_lo * i_lo + u_hi * i_hi
            score = lax.reduce_sum(prod, (0,))
            k = jnp.full((L,), 0, jnp.int32) + (w * WAVE + s)
            plsc.store_scatter(out_v, [k],
                               jnp.full((L,), 0.0, jnp.float32) + score,
                               mask=iota == 0)

    for w0 in range(DEPTH - 1):
        fire(w0)

    def body(w, carry):
        @pl.when(w + DEPTH - 1 < WPT)
        def _():
            fire(w + DEPTH - 1)
        wait_wave()
        compute(w)
        return carry

    lax.fori_loop(0, WPT, body, 0)

    pltpu.sync_copy(out_v, out_hbm.at[pl.ds(base, BPW)])


@jax.jit
def _run(user_ids, item_ids, user_table_t, item_table_t):
    k = pl.kernel(
        _sc_body,
        out_type=jax.ShapeDtypeStruct((B,), jnp.float32),
        mesh=plsc.VectorSubcoreMesh(core_axis_name="c", subcore_axis_name="s"),
        compiler_params=pltpu.CompilerParams(needs_layout_passes=False),
        scratch_types=[
            pltpu.VMEM((BPW,), jnp.int32),
            pltpu.VMEM((BPW,), jnp.int32),
            pltpu.VMEM((DEPTH, WAVE, D, 128), jnp.float32),
            pltpu.VMEM((DEPTH, WAVE, D, 128), jnp.float32),
            pltpu.VMEM((BPW,), jnp.float32),
            pltpu.SemaphoreType.DMA,
            pltpu.SemaphoreType.DMA,
        ],
    )
    return k(user_ids, item_ids, user_table_t, item_table_t)


def kernel(user_ids, item_ids, user_table, item_table):
    return _run(user_ids, item_ids, user_table.T, item_table.T)


# speedup vs baseline: 4.4062x; 1.0005x over previous
import functools

import jax
import jax.numpy as jnp
from jax import lax
from jax.experimental import pallas as pl
from jax.experimental.pallas import tpu as pltpu
from jax.experimental.pallas import tpu_sc as plsc

L = 16
D = 32
B = 16384
NC = 2
NS = 16
NW = NC * NS
BPW = B // NW
WAVE = 2
WPT = BPW // WAVE
DEPTH = 7
WPG = L // WAVE


def _sc_body(uid_hbm, iid_hbm, ut_hbm, it_hbm, out_hbm,
             uidx_v, iidx_v, ubuf_v, ibuf_v, out_v, usem, isem):
    wid = lax.axis_index("s") * NC + lax.axis_index("c")
    base = wid * BPW

    pltpu.sync_copy(uid_hbm.at[pl.ds(base, BPW)], uidx_v)
    pltpu.sync_copy(iid_hbm.at[pl.ds(base, BPW)], iidx_v)

    iota = lax.iota(jnp.int32, L)

    def vecs(w):
        g = (w // WPG) * L
        return uidx_v[pl.ds(g, L)], iidx_v[pl.ds(g, L)]

    def extract(vec, l):
        return lax.reduce_max(jnp.where(iota == l, vec, 0), (0,))

    def fire(w):
        uvec, ivec = vecs(w)
        p = w % DEPTH
        for s in range(WAVE):
            l = (w % WPG) * WAVE + s
            for vec, tab, buf, sem in ((uvec, ut_hbm, ubuf_v, usem),
                                       (ivec, it_hbm, ibuf_v, isem)):
                sid = extract(vec, l)
                jb = pl.multiple_of((sid >> 7) << 7, 128)
                for db in range(4):
                    pltpu.async_copy(tab.at[pl.ds(db * 8, 8), pl.ds(jb, 128)],
                                     buf.at[p, s, pl.ds(db * 8, 8)], sem)

    def wait_wave():
        for s in range(WAVE):
            for db in range(4):
                pltpu.make_async_copy(ut_hbm.at[pl.ds(db * 8, 8),
                                                pl.ds(0, 128)],
                                      ubuf_v.at[0, s, pl.ds(db * 8, 8)],
                                      usem).wait()
                pltpu.make_async_copy(it_hbm.at[pl.ds(db * 8, 8),
                                                pl.ds(0, 128)],
                                      ibuf_v.at[0, s, pl.ds(db * 8, 8)],
                                      isem).wait()

    d_lo = iota
    d_hi = iota + L

    def compute(w):
        uvec, ivec = vecs(w)
        p = w % DEPTH
        pb = jnp.full((L,), 0, jnp.int32) + p
        for s in range(WAVE):
            l = (w % WPG) * WAVE + s
            sb = jnp.full((L,), s, jnp.int32)
            usid = extract(uvec, l)
            isid = extract(ivec, l)
            ulane = jnp.full((L,), 0, jnp.int32) + (usid & 127)
            ilane = jnp.full((L,), 0, jnp.int32) + (isid & 127)
            u_lo = plsc.load_gather(ubuf_v, [pb, sb, d_lo, ulane])
            u_hi = plsc.load_gather(ubuf_v, [pb, sb, d_hi, ulane])
            i_lo = plsc.load_gather(ibuf_v, [pb, sb, d_lo, ilane])
            i_hi = plsc.load_gather(ibuf_v, [pb, sb, d_hi, ilane])
            prod = u_lo * i_lo + u_hi * i_hi
            score = lax.reduce_sum(prod, (0,))
            k = jnp.full((L,), 0, jnp.int32) + (w * WAVE + s)
            plsc.store_scatter(out_v, [k],
                               jnp.full((L,), 0.0, jnp.float32) + score,
                               mask=iota == 0)

    for w0 in range(DEPTH - 1):
        fire(w0)

    def body(w, carry):
        @pl.when(w + DEPTH - 1 < WPT)
        def _():
            fire(w + DEPTH - 1)
        wait_wave()
        compute(w)
        return carry

    lax.fori_loop(0, WPT, body, 0)

    pltpu.sync_copy(out_v, out_hbm.at[pl.ds(base, BPW)])


@jax.jit
def _run(user_ids, item_ids, user_table_t, item_table_t):
    k = pl.kernel(
        _sc_body,
        out_type=jax.ShapeDtypeStruct((B,), jnp.float32),
        mesh=plsc.VectorSubcoreMesh(core_axis_name="c", subcore_axis_name="s"),
        compiler_params=pltpu.CompilerParams(needs_layout_passes=False),
        scratch_types=[
            pltpu.VMEM((BPW,), jnp.int32),
            pltpu.VMEM((BPW,), jnp.int32),
            pltpu.VMEM((DEPTH, WAVE, D, 128), jnp.float32),
            pltpu.VMEM((DEPTH, WAVE, D, 128), jnp.float32),
            pltpu.VMEM((BPW,), jnp.float32),
            pltpu.SemaphoreType.DMA,
            pltpu.SemaphoreType.DMA,
        ],
    )
    return k(user_ids, item_ids, user_table_t, item_table_t)


def kernel(user_ids, item_ids, user_table, item_table):
    return _run(user_ids, item_ids, user_table.T, item_table.T)


# speedup vs baseline: 4.4146x; 1.0019x over previous
import functools

import jax
import jax.numpy as jnp
from jax import lax
from jax.experimental import pallas as pl
from jax.experimental.pallas import tpu as pltpu
from jax.experimental.pallas import tpu_sc as plsc

L = 16
D = 32
B = 16384
NC = 2
NS = 16
NW = NC * NS
BPW = B // NW
WAVE = 2
WPT = BPW // WAVE
DEPTH = 7
WPG = L // WAVE


def _sc_body(uid_hbm, iid_hbm, ut_hbm, it_hbm, out_hbm,
             uidx_v, iidx_v, ubuf_v, ibuf_v, out_v, usem, isem):
    wid = lax.axis_index("s") * NC + lax.axis_index("c")
    base = wid * BPW

    pltpu.sync_copy(uid_hbm.at[pl.ds(base, BPW)], uidx_v)
    pltpu.sync_copy(iid_hbm.at[pl.ds(base, BPW)], iidx_v)

    iota = lax.iota(jnp.int32, L)

    def vecs(w):
        g = (w // WPG) * L
        return uidx_v[pl.ds(g, L)], iidx_v[pl.ds(g, L)]

    def extract(vec, l):
        return lax.reduce_max(jnp.where(iota == l, vec, 0), (0,))

    def fire(w):
        uvec, ivec = vecs(w)
        p = w % DEPTH
        for s in range(WAVE):
            l = (w % WPG) * WAVE + s
            for vec, tab, buf, sem in ((uvec, ut_hbm, ubuf_v, usem),
                                       (ivec, it_hbm, ibuf_v, isem)):
                sid = extract(vec, l)
                jb = pl.multiple_of((sid >> 7) << 7, 128)
                pltpu.async_copy(tab.at[:, pl.ds(jb, 128)], buf.at[p, s], sem)

    def wait_wave():
        for s in range(WAVE):
            pltpu.make_async_copy(ut_hbm.at[:, pl.ds(0, 128)],
                                  ubuf_v.at[0, s], usem).wait()
            pltpu.make_async_copy(it_hbm.at[:, pl.ds(0, 128)],
                                  ibuf_v.at[0, s], isem).wait()

    d_lo = iota
    d_hi = iota + L

    def compute(w):
        uvec, ivec = vecs(w)
        p = w % DEPTH
        pb = jnp.full((L,), 0, jnp.int32) + p
        for s in range(WAVE):
            l = (w % WPG) * WAVE + s
            sb = jnp.full((L,), s, jnp.int32)
            usid = extract(uvec, l)
            isid = extract(ivec, l)
            ulane = jnp.full((L,), 0, jnp.int32) + (usid & 127)
            ilane = jnp.full((L,), 0, jnp.int32) + (isid & 127)
            u_lo = plsc.load_gather(ubuf_v, [pb, sb, d_lo, ulane])
            u_hi = plsc.load_gather(ubuf_v, [pb, sb, d_hi, ulane])
            i_lo = plsc.load_gather(ibuf_v, [pb, sb, d_lo, ilane])
            i_hi = plsc.load_gather(ibuf_v, [pb, sb, d_hi, ilane])
            prod = u_lo * i_lo + u_hi * i_hi
            score = lax.reduce_sum(prod, (0,))
            k = jnp.full((L,), 0, jnp.int32) + (w * WAVE + s)
            plsc.store_scatter(out_v, [k],
                               jnp.full((L,), 0.0, jnp.float32) + score,
                               mask=iota == 0)

    for w0 in range(DEPTH - 1):
        fire(w0)

    def body(w, carry):
        @pl.when(w + DEPTH - 1 < WPT)
        def _():
            fire(w + DEPTH - 1)
        wait_wave()
        compute(w)
        return carry

    lax.fori_loop(0, WPT, body, 0)

    pltpu.sync_copy(out_v, out_hbm.at[pl.ds(base, BPW)])


@jax.jit
def _run(user_ids, item_ids, user_table_t, item_table_t):
    k = pl.kernel(
        _sc_body,
        out_type=jax.ShapeDtypeStruct((B,), jnp.float32),
        mesh=plsc.VectorSubcoreMesh(core_axis_name="c", subcore_axis_name="s"),
        compiler_params=pltpu.CompilerParams(needs_layout_passes=False),
        scratch_types=[
            pltpu.VMEM((BPW,), jnp.int32),
            pltpu.VMEM((BPW,), jnp.int32),
            pltpu.VMEM((DEPTH, WAVE, D, 128), jnp.float32),
            pltpu.VMEM((DEPTH, WAVE, D, 128), jnp.float32),
            pltpu.VMEM((BPW,), jnp.float32),
            pltpu.SemaphoreType.DMA,
            pltpu.SemaphoreType.DMA,
        ],
    )
    return k(user_ids, item_ids, user_table_t, item_table_t)


def kernel(user_ids, item_ids, user_table, item_table):
    return _run(user_ids, item_ids, user_table.T, item_table.T)
